# baseline (speedup 1.0000x reference)
import jax
import jax.numpy as jnp
from jax.experimental import pallas as pl


def _elu(v):
    return jnp.where(v > 0, v, jnp.exp(jnp.minimum(v, 0.0)) - 1.0)


def _backbone_kernel(x_ref, Al_ref, Ar_ref, W1_ref, b1_ref, W2_ref, b2_ref,
                     Wout_ref, bout_ref, out_ref):
    bpb = Al_ref.shape[0]
    half = Al_ref.shape[2]
    W1 = W1_ref[...]
    J = range(bpb)

    def dot(a, b):
        return jnp.dot(a, b, preferred_element_type=jnp.float32)

    Al = [Al_ref[j].astype(jnp.bfloat16) for j in J]
    Ar = [Ar_ref[j].astype(jnp.bfloat16) for j in J]
    z = [x_ref[j] for j in J]

    def hop(j, v):
        vb = v.astype(jnp.bfloat16)
        return dot(Al[j], vb[:half]) + dot(Ar[j], vb[half:])

    acc = [dot(z[j], W1[0:1]) for j in J]
    for i in range(1, 5):
        z = [hop(j, z[j]) for j in J]
        acc = [acc[j] + dot(z[j], W1[i:i + 1]) for j in J]
    h = [_elu(acc[j] + b1_ref[...]) for j in J]

    acc2 = [dot(h[j], W2_ref[0]) for j in J]
    y = [hop(j, h[j]) for j in J]
    acc2 = [acc2[j] + dot(y[j], W2_ref[1]) for j in J]
    y = [hop(j, y[j]) for j in J]
    acc2 = [acc2[j] + dot(y[j], W2_ref[2]) for j in J]
    h2 = [_elu(acc2[j] + b2_ref[...]) for j in J]

    for j in J:
        m = jnp.mean(h2[j], axis=0, keepdims=True)
        out_ref[j] = dot(m, Wout_ref[...]) + bout_ref[...]


def kernel(x, A, W1, b1, W2, b2, Wout, bout):
    B, N, _ = x.shape
    hidden = W2.shape[-1]
    nclass = Wout.shape[-1]

    W1r = W1.reshape(W1.shape[0], hidden)
    b1r = b1.reshape(1, hidden)
    b2r = b2.reshape(1, hidden)
    boutr = bout.reshape(1, nclass)

    bpb = 2
    out = pl.pallas_call(
        _backbone_kernel,
        grid=(B // bpb,),
        in_specs=[
            pl.BlockSpec((bpb, N, 1), lambda b: (b, 0, 0)),
            pl.BlockSpec((bpb, N, N // 2), lambda b: (b, 0, 0)),
            pl.BlockSpec((bpb, N, N // 2), lambda b: (b, 0, 1)),
            pl.BlockSpec(W1r.shape, lambda b: (0, 0)),
            pl.BlockSpec(b1r.shape, lambda b: (0, 0)),
            pl.BlockSpec(W2.shape, lambda b: (0, 0, 0)),
            pl.BlockSpec(b2r.shape, lambda b: (0, 0)),
            pl.BlockSpec(Wout.shape, lambda b: (0, 0)),
            pl.BlockSpec(boutr.shape, lambda b: (0, 0)),
        ],
        out_specs=pl.BlockSpec((bpb, 1, nclass), lambda b: (b, 0, 0)),
        out_shape=jax.ShapeDtypeStruct((B, 1, nclass), jnp.float32),
    )(x, A, A, W1r, b1r, W2, b2r, Wout, boutr)
    return out.reshape(B, nclass)


# speedup vs baseline: 1.0020x; 1.0020x over previous
import jax
import jax.numpy as jnp
from jax.experimental import pallas as pl
from jax.experimental.pallas import tpu as pltpu


def _elu(v):
    return jnp.where(v > 0, v, jnp.exp(jnp.minimum(v, 0.0)) - 1.0)


def _backbone_kernel(x_ref, Al_ref, Ar_ref, W1_ref, b1_ref, W2_ref, b2_ref,
                     Wout_ref, bout_ref, out_ref):
    bpb = Al_ref.shape[0]
    half = Al_ref.shape[2]
    W1 = W1_ref[...]
    J = range(bpb)

    def dot(a, b):
        return jnp.dot(a, b, preferred_element_type=jnp.float32)

    Al = [Al_ref[j].astype(jnp.bfloat16) for j in J]
    Ar = [Ar_ref[j].astype(jnp.bfloat16) for j in J]
    z = [x_ref[j] for j in J]

    def hop(j, v):
        vb = v.astype(jnp.bfloat16)
        return dot(Al[j], vb[:half]) + dot(Ar[j], vb[half:])

    acc = [dot(z[j], W1[0:1]) for j in J]
    for i in range(1, 5):
        z = [hop(j, z[j]) for j in J]
        acc = [acc[j] + dot(z[j], W1[i:i + 1]) for j in J]
    h = [_elu(acc[j] + b1_ref[...]) for j in J]

    acc2 = [dot(h[j], W2_ref[0]) for j in J]
    y = [hop(j, h[j]) for j in J]
    acc2 = [acc2[j] + dot(y[j], W2_ref[1]) for j in J]
    y = [hop(j, y[j]) for j in J]
    acc2 = [acc2[j] + dot(y[j], W2_ref[2]) for j in J]
    h2 = [_elu(acc2[j] + b2_ref[...]) for j in J]

    for j in J:
        m = jnp.mean(h2[j], axis=0, keepdims=True)
        out_ref[j] = dot(m, Wout_ref[...]) + bout_ref[...]


def kernel(x, A, W1, b1, W2, b2, Wout, bout):
    B, N, _ = x.shape
    hidden = W2.shape[-1]
    nclass = Wout.shape[-1]

    W1r = W1.reshape(W1.shape[0], hidden)
    b1r = b1.reshape(1, hidden)
    b2r = b2.reshape(1, hidden)
    boutr = bout.reshape(1, nclass)

    bpb = 2
    out = pl.pallas_call(
        _backbone_kernel,
        grid=(B // bpb,),
        in_specs=[
            pl.BlockSpec((bpb, N, 1), lambda b: (b, 0, 0)),
            pl.BlockSpec((bpb, N, N // 2), lambda b: (b, 0, 0)),
            pl.BlockSpec((bpb, N, N // 2), lambda b: (b, 0, 1)),
            pl.BlockSpec(W1r.shape, lambda b: (0, 0)),
            pl.BlockSpec(b1r.shape, lambda b: (0, 0)),
            pl.BlockSpec(W2.shape, lambda b: (0, 0, 0)),
            pl.BlockSpec(b2r.shape, lambda b: (0, 0)),
            pl.BlockSpec(Wout.shape, lambda b: (0, 0)),
            pl.BlockSpec(boutr.shape, lambda b: (0, 0)),
        ],
        out_specs=pl.BlockSpec((bpb, 1, nclass), lambda b: (b, 0, 0)),
        out_shape=jax.ShapeDtypeStruct((B, 1, nclass), jnp.float32),
        compiler_params=pltpu.CompilerParams(
            dimension_semantics=("parallel",)),
    )(x, A, A, W1r, b1r, W2, b2r, Wout, boutr)
    return out.reshape(B, nclass)


# speedup vs baseline: 1.1051x; 1.1029x over previous
import jax
import jax.numpy as jnp
from jax.experimental import pallas as pl
from jax.experimental.pallas import tpu as pltpu


def _elu(v):
    return jnp.where(v > 0, v, jnp.exp(jnp.minimum(v, 0.0)) - 1.0)


def _backbone_kernel(x_ref, A_ref, W1_ref, b1_ref, W2_ref, b2_ref,
                     Wout_ref, bout_ref, out_ref):
    bpb = A_ref.shape[0]
    W1 = W1_ref[...]
    J = range(bpb)

    def dot(a, b):
        return jnp.dot(a, b, preferred_element_type=jnp.float32)

    A = [A_ref[j].astype(jnp.bfloat16) for j in J]
    z = [x_ref[j] for j in J]

    acc = [b1_ref[...] + z[j] * W1[0:1] for j in J]
    for i in range(1, 5):
        z = [dot(A[j], z[j].astype(jnp.bfloat16)) for j in J]
        acc = [acc[j] + z[j] * W1[i:i + 1] for j in J]
    h = [_elu(acc[j]) for j in J]

    acc2 = [dot(h[j], W2_ref[0]) for j in J]
    y = [dot(A[j], h[j].astype(jnp.bfloat16)) for j in J]
    acc2 = [acc2[j] + dot(y[j], W2_ref[1]) for j in J]
    y = [dot(A[j], y[j].astype(jnp.bfloat16)) for j in J]
    acc2 = [acc2[j] + dot(y[j], W2_ref[2]) for j in J]
    h2 = [_elu(acc2[j] + b2_ref[...]) for j in J]

    for j in J:
        m = jnp.mean(h2[j], axis=0, keepdims=True)
        out_ref[j] = dot(m, Wout_ref[...]) + bout_ref[...]


def kernel(x, A, W1, b1, W2, b2, Wout, bout):
    B, N, _ = x.shape
    hidden = W2.shape[-1]
    nclass = Wout.shape[-1]

    W1r = W1.reshape(W1.shape[0], hidden)
    b1r = b1.reshape(1, hidden)
    b2r = b2.reshape(1, hidden)
    boutr = bout.reshape(1, nclass)

    bpb = 2
    out = pl.pallas_call(
        _backbone_kernel,
        grid=(B // bpb,),
        in_specs=[
            pl.BlockSpec((bpb, N, 1), lambda b: (b, 0, 0)),
            pl.BlockSpec((bpb, N, N), lambda b: (b, 0, 0)),
            pl.BlockSpec(W1r.shape, lambda b: (0, 0)),
            pl.BlockSpec(b1r.shape, lambda b: (0, 0)),
            pl.BlockSpec(W2.shape, lambda b: (0, 0, 0)),
            pl.BlockSpec(b2r.shape, lambda b: (0, 0)),
            pl.BlockSpec(Wout.shape, lambda b: (0, 0)),
            pl.BlockSpec(boutr.shape, lambda b: (0, 0)),
        ],
        out_specs=pl.BlockSpec((bpb, 1, nclass), lambda b: (b, 0, 0)),
        out_shape=jax.ShapeDtypeStruct((B, 1, nclass), jnp.float32),
        compiler_params=pltpu.CompilerParams(
            dimension_semantics=("parallel",)),
    )(x, A, W1r, b1r, W2, b2r, Wout, boutr)
    return out.reshape(B, nclass)


# speedup vs baseline: 1.1289x; 1.0215x over previous
import functools

import jax
import jax.numpy as jnp
from jax.experimental import pallas as pl
from jax.experimental.pallas import tpu as pltpu


def _elu(v):
    return jnp.where(v > 0, v, jnp.exp(jnp.minimum(v, 0.0)) - 1.0)


def _backbone_kernel(x_ref, A_hbm, W1_ref, b1_ref, W2_ref, b2_ref,
                     Wout_ref, bout_ref, out_ref, Abuf, sem, *, nsteps):
    bpb = x_ref.shape[0]
    W1 = W1_ref[...]
    J = range(bpb)
    step = pl.program_id(0)

    def a_copy(src_step, buf):
        return pltpu.make_async_copy(
            A_hbm.at[pl.ds(src_step * bpb, bpb)], Abuf.at[buf], sem.at[buf])

    @pl.when(step == 0)
    def _():
        a_copy(0, 0).start()

    @pl.when(step + 1 < nsteps)
    def _():
        a_copy(step + 1, (step + 1) % 2).start()

    a_copy(step, step % 2).wait()
    cur = step % 2

    def dot(a, b):
        return jnp.dot(a, b, preferred_element_type=jnp.float32)

    A = [Abuf[cur, j].astype(jnp.bfloat16) for j in J]
    z = [x_ref[j] for j in J]

    acc = [b1_ref[...] + z[j] * W1[0:1] for j in J]
    for i in range(1, 5):
        z = [dot(A[j], z[j].astype(jnp.bfloat16)) for j in J]
        acc = [acc[j] + z[j] * W1[i:i + 1] for j in J]
    h = [_elu(acc[j]) for j in J]

    acc2 = [dot(h[j], W2_ref[0]) for j in J]
    y = [dot(A[j], h[j].astype(jnp.bfloat16)) for j in J]
    acc2 = [acc2[j] + dot(y[j], W2_ref[1]) for j in J]
    y = [dot(A[j], y[j].astype(jnp.bfloat16)) for j in J]
    acc2 = [acc2[j] + dot(y[j], W2_ref[2]) for j in J]
    h2 = [_elu(acc2[j] + b2_ref[...]) for j in J]

    for j in J:
        m = jnp.mean(h2[j], axis=0, keepdims=True)
        out_ref[j] = dot(m, Wout_ref[...]) + bout_ref[...]


def kernel(x, A, W1, b1, W2, b2, Wout, bout):
    B, N, _ = x.shape
    hidden = W2.shape[-1]
    nclass = Wout.shape[-1]

    W1r = W1.reshape(W1.shape[0], hidden)
    b1r = b1.reshape(1, hidden)
    b2r = b2.reshape(1, hidden)
    boutr = bout.reshape(1, nclass)

    bpb = 2
    nsteps = B // bpb
    out = pl.pallas_call(
        functools.partial(_backbone_kernel, nsteps=nsteps),
        grid=(nsteps,),
        in_specs=[
            pl.BlockSpec((bpb, N, 1), lambda b: (b, 0, 0)),
            pl.BlockSpec(memory_space=pltpu.MemorySpace.HBM),
            pl.BlockSpec(W1r.shape, lambda b: (0, 0)),
            pl.BlockSpec(b1r.shape, lambda b: (0, 0)),
            pl.BlockSpec(W2.shape, lambda b: (0, 0, 0)),
            pl.BlockSpec(b2r.shape, lambda b: (0, 0)),
            pl.BlockSpec(Wout.shape, lambda b: (0, 0)),
            pl.BlockSpec(boutr.shape, lambda b: (0, 0)),
        ],
        out_specs=pl.BlockSpec((bpb, 1, nclass), lambda b: (b, 0, 0)),
        out_shape=jax.ShapeDtypeStruct((B, 1, nclass), jnp.float32),
        scratch_shapes=[
            pltpu.VMEM((2, bpb, N, N), jnp.float32),
            pltpu.SemaphoreType.DMA((2,)),
        ],
    )(x, A, W1r, b1r, W2, b2r, Wout, boutr)
    return out.reshape(B, nclass)
